# baseline (speedup 1.0000x reference)
import jax
import jax.numpy as jnp
from jax.experimental import pallas as pl
from jax.experimental.pallas import tpu as pltpu

Q = 32
D = 64
K = 5
B = 8000
NEG = -2.0
BIGI = 2 ** 30


def _body(nblk, q_ref, k_ref, ov_ref, oi_ref, rv_ref, ri_ref):
    pid = pl.program_id(0)

    @pl.when(pid == 0)
    def _init():
        rv_ref[...] = jnp.full((Q, 8), NEG, jnp.float32)
        ri_ref[...] = jnp.full((Q, 8), BIGI, jnp.int32)

    q = q_ref[...]
    qn = q / (jnp.sqrt(jnp.sum(q * q, axis=1, keepdims=True)) + 1e-8)
    kb = k_ref[...]
    nsq = jnp.sum(kb * kb, axis=1, keepdims=True)
    inv = 1.0 / (jnp.sqrt(nsq) + 1e-8)
    kbn = kb * inv
    sims = jax.lax.dot_general(
        qn, kbn, (((1,), (1,)), ((), ())),
        preferred_element_type=jnp.float32)

    lane = jax.lax.broadcasted_iota(jnp.int32, (Q, B), 1)
    base = pid * B
    bv, bi = [], []
    s = sims
    for _ in range(K):
        m = jnp.max(s, axis=1, keepdims=True)
        sel = jnp.min(jnp.where(s == m, lane, BIGI), axis=1, keepdims=True)
        bv.append(m)
        bi.append(sel + base)
        s = jnp.where(lane == sel, NEG, s)
    blk_v = jnp.concatenate(bv, axis=1)
    blk_i = jnp.concatenate(bi, axis=1)

    cv = jnp.concatenate([rv_ref[:, :K], blk_v], axis=1)
    ci = jnp.concatenate([ri_ref[:, :K], blk_i], axis=1)
    pos = jax.lax.broadcasted_iota(jnp.int32, (Q, 2 * K), 1)
    nv, ni = [], []
    for _ in range(K):
        m = jnp.max(cv, axis=1, keepdims=True)
        selp = jnp.min(jnp.where(cv == m, pos, BIGI), axis=1, keepdims=True)
        nv.append(m)
        ni.append(jnp.sum(jnp.where(pos == selp, ci, 0), axis=1, keepdims=True))
        cv = jnp.where(pos == selp, NEG, cv)
    rv_ref[:, :K] = jnp.concatenate(nv, axis=1)
    ri_ref[:, :K] = jnp.concatenate(ni, axis=1)

    @pl.when(pid == nblk - 1)
    def _emit():
        ov_ref[...] = rv_ref[:, :K]
        oi_ref[...] = ri_ref[:, :K]


def _topk_retrieval(queries, keys):
    n = keys.shape[0]
    assert n % B == 0, n
    nblk = n // B
    import functools
    return pl.pallas_call(
        functools.partial(_body, nblk),
        grid=(nblk,),
        in_specs=[
            pl.BlockSpec((Q, D), lambda i: (0, 0)),
            pl.BlockSpec((B, D), lambda i: (i, 0)),
        ],
        out_specs=[
            pl.BlockSpec((Q, K), lambda i: (0, 0)),
            pl.BlockSpec((Q, K), lambda i: (0, 0)),
        ],
        out_shape=[
            jax.ShapeDtypeStruct((Q, K), jnp.float32),
            jax.ShapeDtypeStruct((Q, K), jnp.int32),
        ],
        scratch_shapes=[
            pltpu.VMEM((Q, 8), jnp.float32),
            pltpu.VMEM((Q, 8), jnp.int32),
        ],
        compiler_params=pltpu.CompilerParams(
            dimension_semantics=("arbitrary",)),
    )(queries, keys)


def kernel(queries, keys, k):
    vals, idx = _topk_retrieval(queries, keys)
    return vals, idx


# speedup vs baseline: 1.0624x; 1.0624x over previous
import jax
import jax.numpy as jnp
from jax.experimental import pallas as pl
from jax.experimental.pallas import tpu as pltpu

Q = 32
D = 64
K = 5
B = 8000
NEG = -2.0
BIGI = 2 ** 30


def _body(nblk, q_ref, k_ref, ov_ref, oi_ref, rv_ref, ri_ref):
    pid = pl.program_id(0)

    @pl.when(pid == 0)
    def _init():
        rv_ref[...] = jnp.full((Q, 8), NEG, jnp.float32)
        ri_ref[...] = jnp.full((Q, 8), BIGI, jnp.int32)

    q = q_ref[...]
    qn = q / (jnp.sqrt(jnp.sum(q * q, axis=1, keepdims=True)) + 1e-8)
    kb = k_ref[...]
    raw = jax.lax.dot_general(
        qn, kb, (((1,), (1,)), ((), ())),
        preferred_element_type=jnp.float32)
    kb2 = kb * kb
    nsq = jax.lax.dot_general(
        jnp.ones((1, D), jnp.float32), kb2, (((1,), (1,)), ((), ())),
        preferred_element_type=jnp.float32)
    inv = 1.0 / (jnp.sqrt(nsq) + 1e-8)
    sims = raw * inv

    lane = jax.lax.broadcasted_iota(jnp.int32, (Q, B), 1)
    base = pid * B
    bv, bi = [], []
    s = sims
    for _ in range(K):
        m = jnp.max(s, axis=1, keepdims=True)
        sel = jnp.min(jnp.where(s == m, lane, BIGI), axis=1, keepdims=True)
        bv.append(m)
        bi.append(sel + base)
        s = jnp.where(lane == sel, NEG, s)
    blk_v = jnp.concatenate(bv, axis=1)
    blk_i = jnp.concatenate(bi, axis=1)

    cv = jnp.concatenate([rv_ref[:, :K], blk_v], axis=1)
    ci = jnp.concatenate([ri_ref[:, :K], blk_i], axis=1)
    pos = jax.lax.broadcasted_iota(jnp.int32, (Q, 2 * K), 1)
    nv, ni = [], []
    for _ in range(K):
        m = jnp.max(cv, axis=1, keepdims=True)
        selp = jnp.min(jnp.where(cv == m, pos, BIGI), axis=1, keepdims=True)
        nv.append(m)
        ni.append(jnp.sum(jnp.where(pos == selp, ci, 0), axis=1, keepdims=True))
        cv = jnp.where(pos == selp, NEG, cv)
    rv_ref[:, :K] = jnp.concatenate(nv, axis=1)
    ri_ref[:, :K] = jnp.concatenate(ni, axis=1)

    @pl.when(pid == nblk - 1)
    def _emit():
        ov_ref[...] = rv_ref[:, :K]
        oi_ref[...] = ri_ref[:, :K]


def _topk_retrieval(queries, keys):
    n = keys.shape[0]
    assert n % B == 0, n
    nblk = n // B
    import functools
    return pl.pallas_call(
        functools.partial(_body, nblk),
        grid=(nblk,),
        in_specs=[
            pl.BlockSpec((Q, D), lambda i: (0, 0)),
            pl.BlockSpec((B, D), lambda i: (i, 0)),
        ],
        out_specs=[
            pl.BlockSpec((Q, K), lambda i: (0, 0)),
            pl.BlockSpec((Q, K), lambda i: (0, 0)),
        ],
        out_shape=[
            jax.ShapeDtypeStruct((Q, K), jnp.float32),
            jax.ShapeDtypeStruct((Q, K), jnp.int32),
        ],
        scratch_shapes=[
            pltpu.VMEM((Q, 8), jnp.float32),
            pltpu.VMEM((Q, 8), jnp.int32),
        ],
        compiler_params=pltpu.CompilerParams(
            dimension_semantics=("arbitrary",)),
    )(queries, keys)


def kernel(queries, keys, k):
    vals, idx = _topk_retrieval(queries, keys)
    return vals, idx
